# baseline (speedup 1.0000x reference)
import jax
import jax.numpy as jnp
from jax import lax
from jax.experimental import pallas as pl
from jax.experimental.pallas import tpu as pltpu
from jax.experimental.pallas import tpu_sc as plsc

NC, NS, L = 2, 16, 16
NW = NC * NS
B = 16384
RPW = B // NW
CHUNKS = RPW // L

_OFF_IDX = 0
_OFF_G1 = _OFF_IDX + B * 5
_OFF_G0 = _OFF_G1 + B * 2
_OFF_ET = _OFF_G0 + B * 2
_OFF_WM = _OFF_ET + 256
_PACKED = _OFF_WM + 32

_SQRT_MAGIC = 0x1FBD1DF5


def _sqrt16(q):
    qi = lax.bitcast_convert_type(q, jnp.int32)
    y = lax.bitcast_convert_type(
        _SQRT_MAGIC + lax.shift_right_arithmetic(qi, 1), jnp.float32)
    y = 0.5 * (y + q / y)
    y = 0.5 * (y + q / y)
    return y


def _f32(v):
    return lax.bitcast_convert_type(v, jnp.float32)


def _sc_body(packed_hbm, out_hbm, idx_v, g1_v, g0_v, et_v, wm_v, out_v):
    wid = lax.axis_index("s") * NC + lax.axis_index("c")
    base = wid * RPW
    pltpu.sync_copy(packed_hbm.at[pl.ds(_OFF_IDX + base * 5, RPW * 5)], idx_v)
    pltpu.sync_copy(packed_hbm.at[pl.ds(_OFF_G1 + base * 2, RPW * 2)], g1_v)
    pltpu.sync_copy(packed_hbm.at[pl.ds(_OFF_G0 + base * 2, RPW * 2)], g0_v)
    pltpu.sync_copy(packed_hbm.at[pl.ds(_OFF_ET, 256)], et_v)
    pltpu.sync_copy(packed_hbm.at[pl.ds(_OFF_WM, 32)], wm_v)

    iota = lax.iota(jnp.int32, L)
    wm0 = _f32(wm_v[pl.ds(0, L)])
    wm1 = _f32(wm_v[pl.ds(L, L)])

    def chunk(i, carry):
        row = i * L + iota
        row2 = row * 2
        w1a = _f32(plsc.load_gather(g1_v, [row2]))
        w1b = _f32(plsc.load_gather(g1_v, [row2 + 1]))
        w0a = _f32(plsc.load_gather(g0_v, [row2]))
        w0b = _f32(plsc.load_gather(g0_v, [row2 + 1]))
        c0 = w0a * w1a
        c1 = w0a * w1b
        c2 = w0b * w1a
        c3 = w0b * w1b
        row5 = row * 5
        z = []
        for j in range(5):
            o = plsc.load_gather(idx_v, [row5 + j]) * 2
            e0a = _f32(plsc.load_gather(et_v, [o]))
            e0b = _f32(plsc.load_gather(et_v, [o + 1]))
            e1a = _f32(plsc.load_gather(et_v, [o + 62]))
            e1b = _f32(plsc.load_gather(et_v, [o + 63]))
            e2a = _f32(plsc.load_gather(et_v, [o + 124]))
            e2b = _f32(plsc.load_gather(et_v, [o + 125]))
            e3a = _f32(plsc.load_gather(et_v, [o + 186]))
            e3b = _f32(plsc.load_gather(et_v, [o + 187]))
            z.append((c0 * e0a + c1 * e1a + c2 * e2a + c3 * e3a,
                      c0 * e0b + c1 * e1b + c2 * e2b + c3 * e3b))
        qa, qb = z[0]
        s = []
        for r in range(1, 5):
            dx = qa - z[r][0]
            dy = qb - z[r][1]
            s.append(jnp.exp(-10.0 * _sqrt16(wm0 * dx * dx + wm1 * dy * dy)))
        inv = 1.0 / (s[0] + s[1] + s[2] + s[3])
        for r in range(4):
            plsc.store_scatter(out_v, [row, jnp.full((L,), r, jnp.int32)],
                               s[r] * inv)
        return carry

    lax.fori_loop(0, CHUNKS, chunk, 0)
    pltpu.sync_copy(out_v, out_hbm.at[pl.ds(base, RPW), :])


_rank_sc = pl.kernel(
    _sc_body,
    out_type=jax.ShapeDtypeStruct((B, 4), jnp.float32),
    mesh=plsc.VectorSubcoreMesh(core_axis_name="c", subcore_axis_name="s"),
    compiler_params=pltpu.CompilerParams(
        needs_layout_passes=False, use_tc_tiling_on_sc=False),
    scratch_types=[
        pltpu.VMEM((RPW * 5,), jnp.int32),
        pltpu.VMEM((RPW * 2,), jnp.int32),
        pltpu.VMEM((RPW * 2,), jnp.int32),
        pltpu.VMEM((256,), jnp.int32),
        pltpu.VMEM((32,), jnp.int32),
        pltpu.VMEM((RPW, 4), jnp.float32),
    ],
)


def kernel(given4rank1_stimulus_set, percept_gate_weights_1,
           percept_gate_weights_0, E0, E1, E2, E3, w_mink):
    bits = lambda x: lax.bitcast_convert_type(x, jnp.int32).reshape(-1)
    packed = jnp.concatenate([
        given4rank1_stimulus_set.astype(jnp.int32).reshape(-1),
        bits(percept_gate_weights_1),
        bits(percept_gate_weights_0),
        bits(E0), bits(E1), bits(E2), bits(E3),
        jnp.zeros((8,), jnp.int32),
        bits(jnp.broadcast_to(w_mink[:, None], (2, 16))),
    ])
    return _rank_sc(packed)


# speedup vs baseline: 1.1762x; 1.1762x over previous
import jax
import jax.numpy as jnp
from jax import lax
from jax.experimental import pallas as pl
from jax.experimental.pallas import tpu as pltpu
from jax.experimental.pallas import tpu_sc as plsc

NC, NS, L = 2, 16, 16
NW = NC * NS
B = 16384
RPW = B // NW
CHUNKS = RPW // L

_SQRT_MAGIC = 0x1FBD1DF5


def _sqrt16(q):
    qi = lax.bitcast_convert_type(q, jnp.int32)
    y = lax.bitcast_convert_type(
        _SQRT_MAGIC + lax.shift_right_arithmetic(qi, 1), jnp.float32)
    y = 0.5 * (y + q / y)
    y = 0.5 * (y + q / y)
    return y


TB = 128
NTB = RPW // TB


def _sc_body(idx_hbm, g1_hbm, g0_hbm, ew_hbm, out_hbm,
             idx_v, g1_v, g0_v, ew_v, out_v):
    wid = lax.axis_index("s") * NC + lax.axis_index("c")
    base = wid * RPW
    pltpu.sync_copy(ew_hbm, ew_v)

    iota = lax.iota(jnp.int32, L)
    col = [jnp.full((L,), j, jnp.int32) for j in range(5)]
    zeros, ones = col[0], col[1]
    wm0 = ew_v[pl.ds(256, L)]
    wm1 = ew_v[pl.ds(256 + L, L)]

    def chunk(i, carry):
        row = i * L + iota
        w1a = plsc.load_gather(g1_v, [row, zeros])
        w1b = plsc.load_gather(g1_v, [row, ones])
        w0a = plsc.load_gather(g0_v, [row, zeros])
        w0b = plsc.load_gather(g0_v, [row, ones])
        c0 = w0a * w1a
        c1 = w0a * w1b
        c2 = w0b * w1a
        c3 = w0b * w1b
        z = []
        for j in range(5):
            o = plsc.load_gather(idx_v, [row, col[j]]) * 2
            e0a = plsc.load_gather(ew_v, [o])
            e0b = plsc.load_gather(ew_v, [o + 1])
            e1a = plsc.load_gather(ew_v, [o + 62])
            e1b = plsc.load_gather(ew_v, [o + 63])
            e2a = plsc.load_gather(ew_v, [o + 124])
            e2b = plsc.load_gather(ew_v, [o + 125])
            e3a = plsc.load_gather(ew_v, [o + 186])
            e3b = plsc.load_gather(ew_v, [o + 187])
            z.append((c0 * e0a + c1 * e1a + c2 * e2a + c3 * e3a,
                      c0 * e0b + c1 * e1b + c2 * e2b + c3 * e3b))
        qa, qb = z[0]
        s = []
        for r in range(1, 5):
            dx = qa - z[r][0]
            dy = qb - z[r][1]
            s.append(jnp.exp(-10.0 * _sqrt16(wm0 * dx * dx + wm1 * dy * dy)))
        inv = 1.0 / (s[0] + s[1] + s[2] + s[3])
        for r in range(4):
            plsc.store_scatter(out_v, [row, col[r]], s[r] * inv)
        return carry

    for t in range(NTB):
        tb = base + t * TB
        pltpu.sync_copy(idx_hbm.at[pl.ds(tb, TB), :], idx_v)
        pltpu.sync_copy(g1_hbm.at[pl.ds(tb, TB), :], g1_v)
        pltpu.sync_copy(g0_hbm.at[pl.ds(tb, TB), :], g0_v)
        lax.fori_loop(0, TB // L, chunk, 0)
        pltpu.sync_copy(out_v, out_hbm.at[pl.ds(tb, TB), :])


_rank_sc = pl.kernel(
    _sc_body,
    out_type=jax.ShapeDtypeStruct((B, 4), jnp.float32),
    mesh=plsc.VectorSubcoreMesh(core_axis_name="c", subcore_axis_name="s"),
    compiler_params=pltpu.CompilerParams(
        needs_layout_passes=False, use_tc_tiling_on_sc=True),
    scratch_types=[
        pltpu.VMEM((TB, 5), jnp.int32),
        pltpu.VMEM((TB, 2), jnp.float32),
        pltpu.VMEM((TB, 2), jnp.float32),
        pltpu.VMEM((288,), jnp.float32),
        pltpu.VMEM((TB, 4), jnp.float32),
    ],
)


def kernel(given4rank1_stimulus_set, percept_gate_weights_1,
           percept_gate_weights_0, E0, E1, E2, E3, w_mink):
    ew = jnp.concatenate([
        E0.reshape(-1), E1.reshape(-1), E2.reshape(-1), E3.reshape(-1),
        jnp.zeros((8,), jnp.float32),
        jnp.broadcast_to(w_mink[:, None], (2, 16)).reshape(-1),
    ])
    return _rank_sc(given4rank1_stimulus_set.astype(jnp.int32),
                    percept_gate_weights_1, percept_gate_weights_0, ew)


# speedup vs baseline: 1.3405x; 1.1397x over previous
import jax
import jax.numpy as jnp
from jax import lax
from jax.experimental import pallas as pl
from jax.experimental.pallas import tpu as pltpu
from jax.experimental.pallas import tpu_sc as plsc

NC, NS, L = 2, 16, 16
NW = NC * NS
B = 16384
RPW = B // NW
CHUNKS = RPW // L

_SQRT_MAGIC = 0x1FBD1DF5


def _sqrt16(q):
    qi = lax.bitcast_convert_type(q, jnp.int32)
    y = lax.bitcast_convert_type(
        _SQRT_MAGIC + lax.shift_right_arithmetic(qi, 1), jnp.float32)
    y = 0.5 * (y + q / y)
    y = 0.5 * (y + q / y)
    return y


TB = 64
NTB = RPW // TB


def _sc_body(idx_hbm, g1_hbm, g0_hbm, ew_hbm, out_hbm,
             idx_v0, idx_v1, g1_v0, g1_v1, g0_v0, g0_v1, ew_v,
             out_v0, out_v1, sin0, sin1, sout0, sout1):
    idx_b = [idx_v0, idx_v1]
    g1_b = [g1_v0, g1_v1]
    g0_b = [g0_v0, g0_v1]
    out_b = [out_v0, out_v1]
    sin = [sin0, sin1]
    sout = [sout0, sout1]
    wid = lax.axis_index("s") * NC + lax.axis_index("c")
    base = wid * RPW
    pltpu.sync_copy(ew_hbm, ew_v)

    iota = lax.iota(jnp.int32, L)
    col = [jnp.full((L,), j, jnp.int32) for j in range(5)]
    zeros, ones = col[0], col[1]
    wm0 = ew_v[pl.ds(256, L)]
    wm1 = ew_v[pl.ds(256 + L, L)]

    def make_chunk(idx_v, g1_v, g0_v, out_v):
      def chunk(i, carry):
        row = i * L + iota
        w1a = plsc.load_gather(g1_v, [row, zeros])
        w1b = plsc.load_gather(g1_v, [row, ones])
        w0a = plsc.load_gather(g0_v, [row, zeros])
        w0b = plsc.load_gather(g0_v, [row, ones])
        c0 = w0a * w1a
        c1 = w0a * w1b
        c2 = w0b * w1a
        c3 = w0b * w1b
        z = []
        for j in range(5):
            o = plsc.load_gather(idx_v, [row, col[j]]) * 2
            e0a = plsc.load_gather(ew_v, [o])
            e0b = plsc.load_gather(ew_v, [o + 1])
            e1a = plsc.load_gather(ew_v, [o + 62])
            e1b = plsc.load_gather(ew_v, [o + 63])
            e2a = plsc.load_gather(ew_v, [o + 124])
            e2b = plsc.load_gather(ew_v, [o + 125])
            e3a = plsc.load_gather(ew_v, [o + 186])
            e3b = plsc.load_gather(ew_v, [o + 187])
            z.append((c0 * e0a + c1 * e1a + c2 * e2a + c3 * e3a,
                      c0 * e0b + c1 * e1b + c2 * e2b + c3 * e3b))
        qa, qb = z[0]
        s = []
        for r in range(1, 5):
            dx = qa - z[r][0]
            dy = qb - z[r][1]
            s.append(jnp.exp(-10.0 * _sqrt16(wm0 * dx * dx + wm1 * dy * dy)))
        inv = 1.0 / (s[0] + s[1] + s[2] + s[3])
        for r in range(4):
            plsc.store_scatter(out_v, [row, col[r]], s[r] * inv)
        return carry
      return chunk

    def fire_in(t, slot):
        tb = base + t * TB
        return [
            pltpu.async_copy(idx_hbm.at[pl.ds(tb, TB), :], idx_b[slot],
                             sin[slot]),
            pltpu.async_copy(g1_hbm.at[pl.ds(tb, TB), :], g1_b[slot],
                             sin[slot]),
            pltpu.async_copy(g0_hbm.at[pl.ds(tb, TB), :], g0_b[slot],
                             sin[slot]),
        ]

    in_pend = [None, None]
    out_pend = [None, None]
    in_pend[0] = fire_in(0, 0)
    for t in range(NTB):
        slot = t % 2
        if t + 1 < NTB:
            in_pend[1 - slot] = fire_in(t + 1, 1 - slot)
        for h in in_pend[slot]:
            h.wait()
        if out_pend[slot] is not None:
            out_pend[slot].wait()
        lax.fori_loop(0, TB // L,
                      make_chunk(idx_b[slot], g1_b[slot], g0_b[slot],
                                 out_b[slot]), 0)
        out_pend[slot] = pltpu.async_copy(
            out_b[slot], out_hbm.at[pl.ds(base + t * TB, TB), :], sout[slot])
    for slot in range(2):
        if out_pend[slot] is not None:
            out_pend[slot].wait()


_rank_sc = pl.kernel(
    _sc_body,
    out_type=jax.ShapeDtypeStruct((B, 4), jnp.float32),
    mesh=plsc.VectorSubcoreMesh(core_axis_name="c", subcore_axis_name="s"),
    compiler_params=pltpu.CompilerParams(
        needs_layout_passes=False, use_tc_tiling_on_sc=True),
    scratch_types=[
        pltpu.VMEM((TB, 5), jnp.int32),
        pltpu.VMEM((TB, 5), jnp.int32),
        pltpu.VMEM((TB, 2), jnp.float32),
        pltpu.VMEM((TB, 2), jnp.float32),
        pltpu.VMEM((TB, 2), jnp.float32),
        pltpu.VMEM((TB, 2), jnp.float32),
        pltpu.VMEM((288,), jnp.float32),
        pltpu.VMEM((TB, 4), jnp.float32),
        pltpu.VMEM((TB, 4), jnp.float32),
        pltpu.SemaphoreType.DMA,
        pltpu.SemaphoreType.DMA,
        pltpu.SemaphoreType.DMA,
        pltpu.SemaphoreType.DMA,
    ],
)


def kernel(given4rank1_stimulus_set, percept_gate_weights_1,
           percept_gate_weights_0, E0, E1, E2, E3, w_mink):
    ew = jnp.concatenate([
        E0.reshape(-1), E1.reshape(-1), E2.reshape(-1), E3.reshape(-1),
        jnp.zeros((8,), jnp.float32),
        jnp.broadcast_to(w_mink[:, None], (2, 16)).reshape(-1),
    ])
    return _rank_sc(given4rank1_stimulus_set.astype(jnp.int32),
                    percept_gate_weights_1, percept_gate_weights_0, ew)
